# Initial kernel scaffold; baseline (speedup 1.0000x reference)
import jax
import jax.numpy as jnp
from jax.experimental import pallas as pl


def kernel(ftensor, fflow, fmetric, btensor, bflow, bmetric, t, N):
    raise NotImplementedError("write your pallas kernel here")


# speedup vs baseline: 40.9475x; 40.9475x over previous
import functools

import jax
import jax.numpy as jnp
from jax import lax
from jax.experimental import pallas as pl
from jax.experimental.pallas import tpu as pltpu
from jax.experimental.pallas import tpu_sc as plsc

B, C, H, W = 2, 96, 256, 256
HW = H * W
G = 7
NTILES = 16
RPT = HW // NTILES
CHUNK = 128
P2 = 2048
P3 = 2048


def _prep_body(flow_ref, metric_ref, cw_ref, idx_ref):
    gy = lax.broadcasted_iota(jnp.int32, (H, W), 0).astype(jnp.float32)
    gx = lax.broadcasted_iota(jnp.int32, (H, W), 1).astype(jnp.float32)
    fx = gx + flow_ref[0, 0, 0]
    fy = gy + flow_ref[0, 0, 1]
    x0 = jnp.floor(fx)
    y0 = jnp.floor(fy)
    x0i = x0.astype(jnp.int32)
    y0i = y0.astype(jnp.int32)
    wexp = jnp.exp(metric_ref[0, 0])
    for k in range(4):
        dx, dy = k % 2, k // 2
        xi = x0i + dx
        yi = y0i + dy
        wx = (x0 + 1.0 - fx) if dx == 0 else (fx - x0)
        wy = (y0 + 1.0 - fy) if dy == 0 else (fy - y0)
        valid = (xi >= 0) & (xi < W) & (yi >= 0) & (yi < H)
        cw_ref[0, 0, k] = jnp.where(valid, wx * wy * wexp, 0.0)
        idx_ref[0, 0, k] = jnp.clip(yi, 0, H - 1) * W + jnp.clip(xi, 0, W - 1)


def _prep(flow, metric2):
    return pl.pallas_call(
        _prep_body,
        grid=(B, 2),
        in_specs=[
            pl.BlockSpec((1, 1, 2, H, W), lambda b, d: (b, d, 0, 0, 0)),
            pl.BlockSpec((1, 1, H, W), lambda b, d: (b, d, 0, 0)),
        ],
        out_specs=[
            pl.BlockSpec((1, 1, 4, H, W), lambda b, d: (b, d, 0, 0, 0)),
            pl.BlockSpec((1, 1, 4, H, W), lambda b, d: (b, d, 0, 0, 0)),
        ],
        out_shape=[
            jax.ShapeDtypeStruct((B, 2, 4, H, W), jnp.float32),
            jax.ShapeDtypeStruct((B, 2, 4, H, W), jnp.int32),
        ],
    )(flow, metric2)


def _scale_body(tens_ref, cw_ref, out_ref):
    tbT0 = tens_ref[0, 0].T
    tbT1 = tens_ref[0, 1].T
    for dk in range(8):
        d, k = dk // 4, dk % 4
        w = cw_ref[0, d, k, :]
        tbT = tbT0 if d == 0 else tbT1
        out_ref[0, 0, dk] = tbT * w[:, None]


def _scale(tens, cw):
    return pl.pallas_call(
        _scale_body,
        grid=(B, G, HW // P2),
        in_specs=[
            pl.BlockSpec((1, 2, 16, P2), lambda b, g, p: (b, 0, g, p)),
            pl.BlockSpec((1, 2, 4, P2), lambda b, g, p: (b, 0, 0, p)),
        ],
        out_specs=pl.BlockSpec((1, 1, 8, P2, 16),
                               lambda b, g, p: (b, g, 0, p, 0)),
        out_shape=jax.ShapeDtypeStruct((B, G, 8, HW, 16), jnp.float32),
    )(tens, cw)


SCHUNK = 1024
NSUB = SCHUNK // CHUNK


def _sc_splat_body(scaled_hbm, idx_hbm, zeros_hbm, out_hbm,
                   rows_v, idx_v, zero_v, acc_sh):
    b = lax.axis_index("c")
    s = lax.axis_index("s")
    base = s * RPT
    pltpu.sync_copy(zeros_hbm, zero_v)

    def per_g(g, _):
        def zloop(z, _):
            pltpu.sync_copy(zero_v, acc_sh.at[pl.ds(base + z * SCHUNK, SCHUNK)])
            return 0
        lax.fori_loop(0, RPT // SCHUNK, zloop, 0)
        plsc.subcore_barrier()

        def per_dk(dk, _):
            d = dk // 4
            k = dk % 4

            def per_stage(c2, _):
                pltpu.sync_copy(
                    idx_hbm.at[b, d, k,
                               pl.ds(s * (RPT // CHUNK) + c2 * NSUB, NSUB)],
                    idx_v)
                pltpu.sync_copy(
                    scaled_hbm.at[b, g, dk, pl.ds(base + c2 * SCHUNK, SCHUNK)],
                    rows_v)

                def per_chunk(j, _):
                    pltpu.sync_copy(rows_v.at[pl.ds(j * CHUNK, CHUNK)],
                                    acc_sh.at[idx_v.at[j]], add=True)
                    return 0

                lax.fori_loop(0, NSUB, per_chunk, 0)
                return 0

            lax.fori_loop(0, RPT // SCHUNK, per_stage, 0)
            return 0

        lax.fori_loop(0, 8, per_dk, 0)
        plsc.subcore_barrier()

        def dloop(z, _):
            pltpu.sync_copy(acc_sh.at[pl.ds(base + z * SCHUNK, SCHUNK)],
                            out_hbm.at[b, g, pl.ds(base + z * SCHUNK, SCHUNK)])
            return 0
        lax.fori_loop(0, RPT // SCHUNK, dloop, 0)
        plsc.subcore_barrier()
        return 0

    lax.fori_loop(0, G, per_g, 0)


@functools.cache
def _sc_splat_call():
    mesh = plsc.VectorSubcoreMesh(core_axis_name="c", subcore_axis_name="s")
    return pl.kernel(
        _sc_splat_body,
        out_type=jax.ShapeDtypeStruct((B, G, HW, 16), jnp.float32),
        mesh=mesh,
        scratch_types=[
            pltpu.VMEM((SCHUNK, 16), jnp.float32),
            pltpu.VMEM((NSUB, CHUNK), jnp.int32),
            pltpu.VMEM((SCHUNK, 16), jnp.float32),
            pltpu.VMEM_SHARED((HW, 16), jnp.float32),
        ],
        compiler_params=pltpu.CompilerParams(use_tc_tiling_on_sc=False),
    )


def _norm_body(acc_ref, nacc_ref, out_ref):
    nv = nacc_ref[0, 0, :, 0]
    nv = jnp.where(nv == 0.0, 1.0, nv)
    out_ref[0] = acc_ref[0, 0].T / nv[None, :]


def _norm(acc):
    return pl.pallas_call(
        _norm_body,
        grid=(B, 6, HW // P3),
        in_specs=[
            pl.BlockSpec((1, 1, P3, 16), lambda b, g, p: (b, g, p, 0)),
            pl.BlockSpec((1, 1, P3, 16), lambda b, g, p: (b, 6, p, 0)),
        ],
        out_specs=pl.BlockSpec((1, 16, P3), lambda b, g, p: (b, g, p)),
        out_shape=jax.ShapeDtypeStruct((B, C, HW), jnp.float32),
    )(acc, acc)


def kernel(ftensor, fflow, fmetric, btensor, bflow, bmetric, t, N):
    alpha = (t / N).astype(jnp.float32)
    af = jnp.stack([1.0 - alpha, alpha], axis=1)
    logaf = jnp.log(af)
    flow = jnp.stack([fflow, bflow], axis=1)
    metric2 = (jnp.concatenate([fmetric, bmetric], axis=1)
               + logaf[:, :, None, None])

    cw, idx = _prep(flow, metric2)
    cw = cw.reshape(B, 2, 4, HW)
    idx = idx.reshape(B, 2, 4, HW // CHUNK, CHUNK)

    ones = jnp.ones((B, 1, HW), jnp.float32)
    zeros = jnp.zeros((B, 15, HW), jnp.float32)
    ft = jnp.concatenate([ftensor.reshape(B, C, HW), ones, zeros], axis=1)
    bt = jnp.concatenate([btensor.reshape(B, C, HW), ones, zeros], axis=1)
    tens = jnp.stack([ft, bt], axis=1)

    scaled = _scale(tens, cw)

    acc = _sc_splat_call()(scaled, idx, jnp.zeros((SCHUNK, 16), jnp.float32))

    out = _norm(acc)
    return out.reshape(B, C, H, W)


# speedup vs baseline: 130.6779x; 3.1913x over previous
import functools

import jax
import jax.numpy as jnp
from jax import lax
from jax.experimental import pallas as pl
from jax.experimental.pallas import tpu as pltpu
from jax.experimental.pallas import tpu_sc as plsc

B, C, H, W = 2, 96, 256, 256
HW = H * W
G = 7
NTILES = 16
RPT = HW // NTILES
CHUNK = 128
SCHUNK = 512
NSUB = SCHUNK // CHUNK
P2 = 2048
P3 = 2048
UNROLL = 16


def _prep_body(flow_ref, metric_ref, cw_ref, idx_ref):
    gy = lax.broadcasted_iota(jnp.int32, (H, W), 0).astype(jnp.float32)
    gx = lax.broadcasted_iota(jnp.int32, (H, W), 1).astype(jnp.float32)
    fx = gx + flow_ref[0, 0, 0]
    fy = gy + flow_ref[0, 0, 1]
    x0 = jnp.floor(fx)
    y0 = jnp.floor(fy)
    x0i = x0.astype(jnp.int32)
    y0i = y0.astype(jnp.int32)
    wexp = jnp.exp(metric_ref[0, 0])
    for k in range(4):
        dx, dy = k % 2, k // 2
        xi = x0i + dx
        yi = y0i + dy
        wx = (x0 + 1.0 - fx) if dx == 0 else (fx - x0)
        wy = (y0 + 1.0 - fy) if dy == 0 else (fy - y0)
        valid = (xi >= 0) & (xi < W) & (yi >= 0) & (yi < H)
        cw_ref[0, 0, k] = jnp.where(valid, wx * wy * wexp, 0.0)
        idx_ref[0, 0, k] = jnp.clip(yi, 0, H - 1) * W + jnp.clip(xi, 0, W - 1)


def _prep(flow, metric2):
    return pl.pallas_call(
        _prep_body,
        grid=(B, 2),
        in_specs=[
            pl.BlockSpec((1, 1, 2, H, W), lambda b, d: (b, d, 0, 0, 0)),
            pl.BlockSpec((1, 1, H, W), lambda b, d: (b, d, 0, 0)),
        ],
        out_specs=[
            pl.BlockSpec((1, 1, 4, H, W), lambda b, d: (b, d, 0, 0, 0)),
            pl.BlockSpec((1, 1, 4, H, W), lambda b, d: (b, d, 0, 0, 0)),
        ],
        out_shape=[
            jax.ShapeDtypeStruct((B, 2, 4, H, W), jnp.float32),
            jax.ShapeDtypeStruct((B, 2, 4, H, W), jnp.int32),
        ],
    )(flow, metric2)


def _pm_body(tens_ref, out_ref):
    pad = jnp.where(
        lax.broadcasted_iota(jnp.int32, (32, P2), 0) == 0, 1.0, 0.0)
    full = jnp.concatenate([tens_ref[0, 0], pad], axis=0)
    out_ref[0, 0] = full.T


def _pixel_major(tens):
    return pl.pallas_call(
        _pm_body,
        grid=(B, 2, HW // P2),
        in_specs=[pl.BlockSpec((1, 1, C, P2), lambda b, d, p: (b, d, 0, p))],
        out_specs=pl.BlockSpec((1, 1, P2, 128), lambda b, d, p: (b, d, p, 0)),
        out_shape=jax.ShapeDtypeStruct((B, 2, HW, 128), jnp.float32),
    )(tens)


def _sc_splat_body(tpm_hbm, cw_hbm, idx_hbm, zeros_hbm, out_hbm,
                   rows_v, sb0, sb1, w_v, idx_v, zero_v, acc_sh,
                   sem0, sem1):
    b = lax.axis_index("c")
    s = lax.axis_index("s")
    base = s * RPT
    pltpu.sync_copy(zeros_hbm, zero_v)
    sbufs = (sb0, sb1)
    sems = (sem0, sem1)

    def per_g(g, _):
        def zloop(z, _):
            pltpu.sync_copy(zero_v, acc_sh.at[pl.ds(base + z * SCHUNK, SCHUNK)])
            return 0
        lax.fori_loop(0, RPT // SCHUNK, zloop, 0)
        plsc.subcore_barrier()

        def per_dc(dc, _):
            d = dc // (RPT // SCHUNK)
            c2 = dc % (RPT // SCHUNK)
            off = base + c2 * SCHUNK
            pltpu.sync_copy(
                tpm_hbm.at[b, d, pl.ds(off, SCHUNK), pl.ds(g * 16, 16)],
                rows_v)
            pltpu.sync_copy(cw_hbm.at[b, d, :, pl.ds(off, SCHUNK)], w_v)
            pltpu.sync_copy(
                idx_hbm.at[b, d, :, pl.ds(off // CHUNK, NSUB)], idx_v)

            pend = [None, None]
            for k in range(4):
                sb = sbufs[k % 2]
                if pend[k % 2] is not None:
                    for dsc in pend[k % 2]:
                        dsc.wait()

                def scale16(i, _, sb=sb, k=k):
                    r = i * UNROLL
                    wv = w_v[k, pl.ds(r, UNROLL)]
                    for u in range(UNROLL):
                        sb[r + u, :] = rows_v[r + u, :] * wv[u]
                    return 0
                lax.fori_loop(0, SCHUNK // UNROLL, scale16, 0)

                pend[k % 2] = [
                    pltpu.async_copy(sb.at[pl.ds(j * CHUNK, CHUNK)],
                                     acc_sh.at[idx_v.at[k, j]],
                                     sems[k % 2], add=True)
                    for j in range(NSUB)
                ]

            for q in range(2):
                for dsc in pend[q]:
                    dsc.wait()
            return 0

        lax.fori_loop(0, 2 * (RPT // SCHUNK), per_dc, 0)
        plsc.subcore_barrier()

        def dloop(z, _):
            pltpu.sync_copy(acc_sh.at[pl.ds(base + z * SCHUNK, SCHUNK)],
                            out_hbm.at[b, g, pl.ds(base + z * SCHUNK, SCHUNK)])
            return 0
        lax.fori_loop(0, RPT // SCHUNK, dloop, 0)
        plsc.subcore_barrier()
        return 0

    lax.fori_loop(0, G, per_g, 0)


@functools.cache
def _sc_splat_call():
    mesh = plsc.VectorSubcoreMesh(core_axis_name="c", subcore_axis_name="s")
    return pl.kernel(
        _sc_splat_body,
        out_type=jax.ShapeDtypeStruct((B, G, HW, 16), jnp.float32),
        mesh=mesh,
        scratch_types=[
            pltpu.VMEM((SCHUNK, 16), jnp.float32),
            pltpu.VMEM((SCHUNK, 16), jnp.float32),
            pltpu.VMEM((SCHUNK, 16), jnp.float32),
            pltpu.VMEM((4, SCHUNK), jnp.float32),
            pltpu.VMEM((4, NSUB, CHUNK), jnp.int32),
            pltpu.VMEM((SCHUNK, 16), jnp.float32),
            pltpu.VMEM_SHARED((HW, 16), jnp.float32),
            pltpu.SemaphoreType.DMA,
            pltpu.SemaphoreType.DMA,
        ],
        compiler_params=pltpu.CompilerParams(use_tc_tiling_on_sc=False),
    )


def _norm_body(acc_ref, nacc_ref, out_ref):
    nvT = nacc_ref[0, 0].T
    nv = nvT[0:1]
    nv = jnp.where(nv == 0.0, 1.0, nv)
    out_ref[0] = acc_ref[0, 0].T / nv


def _norm(acc):
    return pl.pallas_call(
        _norm_body,
        grid=(B, 6, HW // P3),
        in_specs=[
            pl.BlockSpec((1, 1, P3, 16), lambda b, g, p: (b, g, p, 0)),
            pl.BlockSpec((1, 1, P3, 16), lambda b, g, p: (b, 6, p, 0)),
        ],
        out_specs=pl.BlockSpec((1, 16, P3), lambda b, g, p: (b, g, p)),
        out_shape=jax.ShapeDtypeStruct((B, C, HW), jnp.float32),
    )(acc, acc)


def kernel(ftensor, fflow, fmetric, btensor, bflow, bmetric, t, N):
    alpha = (t / N).astype(jnp.float32)
    af = jnp.stack([1.0 - alpha, alpha], axis=1)
    logaf = jnp.log(af)
    flow = jnp.stack([fflow, bflow], axis=1)
    metric2 = (jnp.concatenate([fmetric, bmetric], axis=1)
               + logaf[:, :, None, None])

    cw, idx = _prep(flow, metric2)
    cw = cw.reshape(B, 2, 4, HW)
    idx = idx.reshape(B, 2, 4, HW // CHUNK, CHUNK)

    tens = jnp.stack([ftensor.reshape(B, C, HW),
                      btensor.reshape(B, C, HW)], axis=1)
    tpm = _pixel_major(tens)

    acc = _sc_splat_call()(tpm, cw, idx,
                           jnp.zeros((SCHUNK, 16), jnp.float32))

    out = _norm(acc)
    return out.reshape(B, C, H, W)
